# baseline (speedup 1.0000x reference)
import functools

import jax
import jax.numpy as jnp
import numpy as np
from jax import lax
from jax.experimental import pallas as pl
from jax.experimental.pallas import tpu as pltpu
from jax.experimental.pallas import tpu_sc as plsc

EMBED_DIM = 32
MAX_SIZE = 1000000000
VOCAB = 1000000
BATCH = 16384

NBW = 15625
K_IDX = np.float32(np.log10(2.0) / (9.0 / (VOCAB - 1)))

_POLY = (
    np.float32(-0.024825606495141983),
    np.float32(0.2668588161468506),
    np.float32(-1.2342631816864014),
    np.float32(3.2188327312469482),
    np.float32(-5.264110565185547),
    np.float32(6.065830230712891),
    np.float32(-3.028317451477051),
)


def _window_table() -> np.ndarray:
    b = np.logspace(0.0, np.log10(MAX_SIZE), num=VOCAB).astype(np.float32)
    bpad = np.concatenate(
        [b, np.full(64 * NBW + 128 - VOCAB, np.inf, dtype=np.float32)])
    idx = 64 * np.arange(NBW)[:, None] + np.arange(128)[None, :]
    return bpad[idx]


_BWIN = _window_table()

_NC, _NS, _L = 2, 16, 16
_NW = _NC * _NS
_BPW = BATCH // _NW
_CHUNK = 128
_NCHUNK = _BPW // _CHUNK
_RING = 4

_NBLK = 224
_BPS = _NBLK // _NS
_W = _NBLK * 128
_R1 = 999936
_R0 = _R1 - _W
_SPROWS = _NBLK * 32


def _sc_body(x_hbm, bwin_hbm, tabt_hbm, out_hbm,
             sp, x_v, blk_v, b0_v, win_v, t_v, idxr_v,
             tbufa, tbufb, buf2, ring_v, rowb_v, obuf_v, sem):
    wid = lax.axis_index("s") * _NC + lax.axis_index("c")
    sid = lax.axis_index("s")
    base = wid * _BPW
    iota = lax.iota(jnp.int32, _L)
    d_lo = iota
    d_hi = iota + _L

    pltpu.sync_copy(x_hbm.at[pl.ds(base, _BPW)], x_v)

    def transpose_block(src_ref, blk):
        def tr_row(r, carry):
            for jj in range(8):
                jv = jj * _L + iota
                dv = jv & 31
                cv = 4 * r + (jv >> 5)
                w = plsc.load_gather(src_ref, [dv, cv])
                buf2[r, pl.ds(jj * _L, _L)] = w
            return carry

        lax.fori_loop(0, 32, tr_row, 0, unroll=False)
        pltpu.sync_copy(
            buf2, sp.at[pl.ds(pl.multiple_of(blk * 32, 8), 32), :])

    def stage_pair(k, carry):
        blk0 = sid * _BPS + 2 * k
        col0 = pl.multiple_of(_R0 + blk0 * 128, 128)
        ca = pltpu.async_copy(
            tabt_hbm.at[pl.ds(0, EMBED_DIM), pl.ds(col0, 128)], tbufa, sem)
        cb = pltpu.async_copy(
            tabt_hbm.at[pl.ds(0, EMBED_DIM),
                        pl.ds(pl.multiple_of(col0 + 128, 128), 128)],
            tbufb, sem)
        ca.wait()
        transpose_block(tbufa, blk0)
        cb.wait()
        transpose_block(tbufb, blk0 + 1)
        return carry

    lax.fori_loop(0, _BPS // 2, stage_pair, 0, unroll=False)
    plsc.subcore_barrier()

    def chunk_loop(fc, carry):
        e0 = fc * _CHUNK

        def guess_group(i, carry2):
            x = x_v[pl.ds(e0 + i * _L, _L)]
            bits = lax.bitcast_convert_type(x, jnp.int32)
            e = (bits >> 23) - 127
            m = lax.bitcast_convert_type((bits & 0x7FFFFF) | 0x3F800000,
                                         jnp.float32)
            acc = jnp.full((_L,), _POLY[0], dtype=jnp.float32)
            for c in _POLY[1:]:
                acc = acc * m + c
            log2x = e.astype(jnp.float32) + acc
            g = (log2x * K_IDX).astype(jnp.int32) + 1
            b0 = jnp.clip(g - 8, 0, VOCAB - 17)
            blk_v[pl.ds(i * _L, _L)] = b0 >> 6
            b0_v[pl.ds(i * _L, _L)] = b0
            return carry2

        lax.fori_loop(0, _CHUNK // _L, guess_group, 0, unroll=False)

        pltpu.async_copy(bwin_hbm.at[blk_v], win_v, sem).wait()

        def count_group(i, carry2):
            x = x_v[pl.ds(e0 + i * _L, _L)]
            b0 = b0_v[pl.ds(i * _L, _L)]
            s = b0 & 63
            rows = i * _L + iota
            cnt = jnp.zeros((_L,), jnp.int32)
            for j in range(17):
                w = plsc.load_gather(win_v, [rows, s + j])
                cnt = cnt + jnp.where(w <= x, 1, 0)
            t = jnp.minimum(b0 + cnt, VOCAB - 1)
            t_v[pl.ds(i * _L, _L)] = t
            idxr_v[pl.ds(i * _L, _L)] = jnp.clip(t - _R0, 0, _W - 1) >> 2
            return carry2

        lax.fori_loop(0, _CHUNK // _L, count_group, 0, unroll=False)

        pltpu.async_copy(sp.at[idxr_v], rowb_v, sem).wait()

        def extract_sp(i, carry2):
            tv = t_v[pl.ds(i * _L, _L)]
            cv = jnp.clip(tv - _R0, 0, _W - 1)
            j0v = (cv & 3) * 32
            for j in range(_L):
                e_loc = i * _L + j
                j0 = j0v[j]
                ee = jnp.full((_L,), e_loc, jnp.int32)
                lo = plsc.load_gather(rowb_v, [ee, j0 + d_lo])
                hi = plsc.load_gather(rowb_v, [ee, j0 + d_hi])
                plsc.store_scatter(obuf_v, [d_lo, ee], lo)
                plsc.store_scatter(obuf_v, [d_hi, ee], hi)
            return carry2

        lax.fori_loop(0, _CHUNK // _L, extract_sp, 0, unroll=False)

        def fallback_group(i, carry2):
            tv = t_v[pl.ds(i * _L, _L)]
            for h in (0, 4, 8, 12):
                for j in range(h, h + 4):
                    t = tv[j]
                    fb = jnp.logical_or(t < _R0, t >= _R1)

                    @pl.when(fb)
                    def _(t=t, slot=j - h):
                        col = pl.multiple_of((t >> 7) * 128, 128)
                        pltpu.async_copy(
                            tabt_hbm.at[pl.ds(0, EMBED_DIM),
                                        pl.ds(col, 128)],
                            ring_v.at[slot], sem)

                for j in range(h, h + 4):
                    t = tv[j]
                    fb = jnp.logical_or(t < _R0, t >= _R1)

                    @pl.when(fb)
                    def _(t=t, slot=j - h, e_loc=i * _L + j):
                        pltpu.make_async_copy(
                            tabt_hbm.at[pl.ds(0, EMBED_DIM), pl.ds(0, 128)],
                            ring_v.at[slot], sem).wait()
                        off = jnp.full((_L,), t & 127, jnp.int32)
                        ee = jnp.full((_L,), e_loc, jnp.int32)
                        lo = plsc.load_gather(ring_v.at[slot], [d_lo, off])
                        hi = plsc.load_gather(ring_v.at[slot], [d_hi, off])
                        plsc.store_scatter(obuf_v, [d_lo, ee], lo)
                        plsc.store_scatter(obuf_v, [d_hi, ee], hi)
            return carry2

        lax.fori_loop(0, _CHUNK // _L, fallback_group, 0, unroll=False)

        pltpu.sync_copy(
            obuf_v,
            out_hbm.at[pl.ds(0, EMBED_DIM),
                       pl.ds(pl.multiple_of(base + e0, _CHUNK), _CHUNK)])
        return carry

    lax.fori_loop(0, _NCHUNK, chunk_loop, 0, unroll=False)


@jax.jit
def _embed(inputs, bwin, table):
    tabt = table.T
    mesh = plsc.VectorSubcoreMesh(core_axis_name="c", subcore_axis_name="s")
    out_t = pl.kernel(
        _sc_body,
        out_type=jax.ShapeDtypeStruct((EMBED_DIM, BATCH), jnp.float32),
        mesh=mesh,
        scratch_types=[
            pltpu.VMEM_SHARED((_SPROWS, 128), jnp.float32),
            pltpu.VMEM((_BPW,), jnp.float32),
            pltpu.VMEM((_CHUNK,), jnp.int32),
            pltpu.VMEM((_CHUNK,), jnp.int32),
            pltpu.VMEM((_CHUNK, 128), jnp.float32),
            pltpu.VMEM((_CHUNK,), jnp.int32),
            pltpu.VMEM((_CHUNK,), jnp.int32),
            pltpu.VMEM((EMBED_DIM, 128), jnp.float32),
            pltpu.VMEM((EMBED_DIM, 128), jnp.float32),
            pltpu.VMEM((32, 128), jnp.float32),
            pltpu.VMEM((_RING, EMBED_DIM, 128), jnp.float32),
            pltpu.VMEM((_CHUNK, 128), jnp.float32),
            pltpu.VMEM((EMBED_DIM, _CHUNK), jnp.float32),
            pltpu.SemaphoreType.DMA,
        ],
        compiler_params=pltpu.CompilerParams(
            needs_layout_passes=False, use_tc_tiling_on_sc=True),
    )(inputs, bwin, tabt)
    return out_t.T


def kernel(inputs, table):
    bwin = jnp.asarray(_BWIN)
    return _embed(inputs, bwin, table)


# speedup vs baseline: 1.8307x; 1.8307x over previous
import functools

import jax
import jax.numpy as jnp
import numpy as np
from jax import lax
from jax.experimental import pallas as pl
from jax.experimental.pallas import tpu as pltpu
from jax.experimental.pallas import tpu_sc as plsc

EMBED_DIM = 32
MAX_SIZE = 1000000000
VOCAB = 1000000
BATCH = 16384

NBW = 15625
K_IDX = np.float32(np.log10(2.0) / (9.0 / (VOCAB - 1)))

_POLY = (
    np.float32(-0.024825606495141983),
    np.float32(0.2668588161468506),
    np.float32(-1.2342631816864014),
    np.float32(3.2188327312469482),
    np.float32(-5.264110565185547),
    np.float32(6.065830230712891),
    np.float32(-3.028317451477051),
)


def _window_table() -> np.ndarray:
    b = np.logspace(0.0, np.log10(MAX_SIZE), num=VOCAB).astype(np.float32)
    bpad = np.concatenate(
        [b, np.full(64 * NBW + 128 - VOCAB, np.inf, dtype=np.float32)])
    idx = 64 * np.arange(NBW)[:, None] + np.arange(128)[None, :]
    return bpad[idx]


_BWIN = _window_table()

_NC, _NS, _L = 2, 16, 16
_NW = _NC * _NS
_BPW = BATCH // _NW
_GROUPS = _BPW // _L
_CHUNK = 128
_NCHUNK = _BPW // _CHUNK
_RING = 8


def _sc_body(x_hbm, bwin_hbm, tabt_hbm, out_hbm,
             x_v, blk_v, sub_v, win_v, t_v, ring_v, obuf_v, sem):
    wid = lax.axis_index("s") * _NC + lax.axis_index("c")
    base = wid * _BPW

    pltpu.sync_copy(x_hbm.at[pl.ds(base, _BPW)], x_v)

    def guess_group(i, carry):
        x = x_v[pl.ds(i * _L, _L)]
        bits = lax.bitcast_convert_type(x, jnp.int32)
        e = (bits >> 23) - 127
        m = lax.bitcast_convert_type((bits & 0x7FFFFF) | 0x3F800000,
                                     jnp.float32)
        acc = jnp.full((_L,), _POLY[0], dtype=jnp.float32)
        for c in _POLY[1:]:
            acc = acc * m + c
        log2x = e.astype(jnp.float32) + acc
        g = (log2x * K_IDX).astype(jnp.int32) + 1
        b0 = jnp.clip(g - 8, 0, VOCAB - 17)
        blk_v[pl.ds(i * _L, _L)] = b0 >> 6
        sub_v[pl.ds(i * _L, _L)] = b0
        return carry

    lax.fori_loop(0, _GROUPS, guess_group, 0, unroll=False)

    copies = [
        pltpu.async_copy(
            bwin_hbm.at[blk_v.at[pl.ds(j * _CHUNK, _CHUNK)]],
            win_v.at[pl.ds(j * _CHUNK, _CHUNK)],
            sem,
        )
        for j in range(_NCHUNK)
    ]
    for c in copies:
        c.wait()

    def count_group(i, carry):
        x = x_v[pl.ds(i * _L, _L)]
        b0 = sub_v[pl.ds(i * _L, _L)]
        s = b0 & 63
        rows = i * _L + lax.iota(jnp.int32, _L)
        cnt = jnp.zeros((_L,), jnp.int32)
        for j in range(17):
            w = plsc.load_gather(win_v, [rows, s + j])
            cnt = cnt + jnp.where(w <= x, 1, 0)
        t = jnp.minimum(b0 + cnt, VOCAB - 1)
        t_v[pl.ds(i * _L, _L)] = t
        return carry

    lax.fori_loop(0, _GROUPS, count_group, 0, unroll=False)

    d_lo = lax.iota(jnp.int32, _L)
    d_hi = d_lo + _L

    def fetch(tv, j, slot):
        col = (tv[j] >> 7) * 128
        return pltpu.async_copy(
            tabt_hbm.at[pl.ds(0, EMBED_DIM), pl.ds(pl.multiple_of(col, 128),
                                                   128)],
            ring_v.at[slot],
            sem,
        )

    def extract(tv, j, slot, e_loc):
        off = jnp.full((_L,), tv[j] & 127, jnp.int32)
        ee = jnp.full((_L,), e_loc, jnp.int32)
        lo = plsc.load_gather(ring_v.at[slot], [d_lo, off])
        hi = plsc.load_gather(ring_v.at[slot], [d_hi, off])
        plsc.store_scatter(obuf_v, [d_lo, ee], lo)
        plsc.store_scatter(obuf_v, [d_hi, ee], hi)

    def chunk_loop(fc, carry):
        e0 = fc * _CHUNK
        tvs = [t_v[pl.ds(e0 + g * _L, _L)] for g in range(_CHUNK // _L)]
        descs = []
        for j in range(_RING):
            descs.append(fetch(tvs[j // _L], j % _L, j % _RING))
        for j in range(_CHUNK):
            descs[j].wait()
            extract(tvs[j // _L], j % _L, j % _RING, j)
            if j + _RING < _CHUNK:
                jj = j + _RING
                descs.append(fetch(tvs[jj // _L], jj % _L, jj % _RING))
        pltpu.sync_copy(
            obuf_v,
            out_hbm.at[pl.ds(0, EMBED_DIM),
                       pl.ds(pl.multiple_of(base + e0, _CHUNK), _CHUNK)])
        return carry

    lax.fori_loop(0, _NCHUNK, chunk_loop, 0, unroll=False)


@jax.jit
def _embed(inputs, bwin, table):
    tabt = table.T
    mesh = plsc.VectorSubcoreMesh(core_axis_name="c", subcore_axis_name="s")
    out_t = pl.kernel(
        _sc_body,
        out_type=jax.ShapeDtypeStruct((EMBED_DIM, BATCH), jnp.float32),
        mesh=mesh,
        scratch_types=[
            pltpu.VMEM((_BPW,), jnp.float32),
            pltpu.VMEM((_BPW,), jnp.int32),
            pltpu.VMEM((_BPW,), jnp.int32),
            pltpu.VMEM((_BPW, 128), jnp.float32),
            pltpu.VMEM((_BPW,), jnp.int32),
            pltpu.VMEM((_RING, EMBED_DIM, 128), jnp.float32),
            pltpu.VMEM((EMBED_DIM, _CHUNK), jnp.float32),
            pltpu.SemaphoreType.DMA,
        ],
        compiler_params=pltpu.CompilerParams(
            needs_layout_passes=False, use_tc_tiling_on_sc=True),
    )(inputs, bwin, tabt)
    return out_t.T


def kernel(inputs, table):
    bwin = jnp.asarray(_BWIN)
    return _embed(inputs, bwin, table)


# speedup vs baseline: 1.8515x; 1.0114x over previous
import functools

import jax
import jax.numpy as jnp
import numpy as np
from jax import lax
from jax.experimental import pallas as pl
from jax.experimental.pallas import tpu as pltpu
from jax.experimental.pallas import tpu_sc as plsc

EMBED_DIM = 32
MAX_SIZE = 1000000000
VOCAB = 1000000
BATCH = 16384

NBW = 15625
K_IDX = np.float32(np.log10(2.0) / (9.0 / (VOCAB - 1)))

_POLY = (
    np.float32(-0.024825606495141983),
    np.float32(0.2668588161468506),
    np.float32(-1.2342631816864014),
    np.float32(3.2188327312469482),
    np.float32(-5.264110565185547),
    np.float32(6.065830230712891),
    np.float32(-3.028317451477051),
)


def _window_table() -> np.ndarray:
    b = np.logspace(0.0, np.log10(MAX_SIZE), num=VOCAB).astype(np.float32)
    bpad = np.concatenate(
        [b, np.full(64 * NBW + 128 - VOCAB, np.inf, dtype=np.float32)])
    idx = 64 * np.arange(NBW)[:, None] + np.arange(128)[None, :]
    return bpad[idx]


_BWIN = _window_table()

_NC, _NS, _L = 2, 16, 16
_NW = _NC * _NS
_BPW = BATCH // _NW
_GROUPS = _BPW // _L
_CHUNK = 128
_NCHUNK = _BPW // _CHUNK
_RING = 12


def _sc_body(x_hbm, bwin_hbm, tabt_hbm, out_hbm,
             x_v, blk_v, sub_v, win_v, t_v, ring_v, obuf_v, sem):
    wid = lax.axis_index("s") * _NC + lax.axis_index("c")
    base = wid * _BPW

    pltpu.sync_copy(x_hbm.at[pl.ds(base, _BPW)], x_v)

    def guess_group(i, carry):
        x = x_v[pl.ds(i * _L, _L)]
        bits = lax.bitcast_convert_type(x, jnp.int32)
        e = (bits >> 23) - 127
        m = lax.bitcast_convert_type((bits & 0x7FFFFF) | 0x3F800000,
                                     jnp.float32)
        acc = jnp.full((_L,), _POLY[0], dtype=jnp.float32)
        for c in _POLY[1:]:
            acc = acc * m + c
        log2x = e.astype(jnp.float32) + acc
        g = (log2x * K_IDX).astype(jnp.int32) + 1
        b0 = jnp.clip(g - 8, 0, VOCAB - 17)
        blk_v[pl.ds(i * _L, _L)] = b0 >> 6
        sub_v[pl.ds(i * _L, _L)] = b0
        return carry

    lax.fori_loop(0, _GROUPS, guess_group, 0, unroll=False)

    copies = [
        pltpu.async_copy(
            bwin_hbm.at[blk_v.at[pl.ds(j * _CHUNK, _CHUNK)]],
            win_v.at[pl.ds(j * _CHUNK, _CHUNK)],
            sem,
        )
        for j in range(_NCHUNK)
    ]
    for c in copies:
        c.wait()

    def count_group(i, carry):
        x = x_v[pl.ds(i * _L, _L)]
        b0 = sub_v[pl.ds(i * _L, _L)]
        s = b0 & 63
        rows = i * _L + lax.iota(jnp.int32, _L)
        cnt = jnp.zeros((_L,), jnp.int32)
        for j in range(17):
            w = plsc.load_gather(win_v, [rows, s + j])
            cnt = cnt + jnp.where(w <= x, 1, 0)
        t = jnp.minimum(b0 + cnt, VOCAB - 1)
        t_v[pl.ds(i * _L, _L)] = t
        return carry

    lax.fori_loop(0, _GROUPS, count_group, 0, unroll=False)

    d_lo = lax.iota(jnp.int32, _L)
    d_hi = d_lo + _L

    def fetch(tv, j, slot):
        col = (tv[j] >> 7) * 128
        return pltpu.async_copy(
            tabt_hbm.at[pl.ds(0, EMBED_DIM), pl.ds(pl.multiple_of(col, 128),
                                                   128)],
            ring_v.at[slot],
            sem,
        )

    def extract(tv, j, slot, e_loc):
        off = jnp.full((_L,), tv[j] & 127, jnp.int32)
        ee = jnp.full((_L,), e_loc, jnp.int32)
        lo = plsc.load_gather(ring_v.at[slot], [d_lo, off])
        hi = plsc.load_gather(ring_v.at[slot], [d_hi, off])
        plsc.store_scatter(obuf_v, [d_lo, ee], lo)
        plsc.store_scatter(obuf_v, [d_hi, ee], hi)

    def chunk_loop(fc, carry):
        e0 = fc * _CHUNK
        tvs = [t_v[pl.ds(e0 + g * _L, _L)] for g in range(_CHUNK // _L)]
        descs = []
        for j in range(_RING):
            descs.append(fetch(tvs[j // _L], j % _L, j % _RING))
        for j in range(_CHUNK):
            descs[j].wait()
            extract(tvs[j // _L], j % _L, j % _RING, j)
            if j + _RING < _CHUNK:
                jj = j + _RING
                descs.append(fetch(tvs[jj // _L], jj % _L, jj % _RING))
        pltpu.sync_copy(
            obuf_v,
            out_hbm.at[pl.ds(0, EMBED_DIM),
                       pl.ds(pl.multiple_of(base + e0, _CHUNK), _CHUNK)])
        return carry

    lax.fori_loop(0, _NCHUNK, chunk_loop, 0, unroll=False)


@jax.jit
def _embed(inputs, bwin, table):
    tabt = table.T
    mesh = plsc.VectorSubcoreMesh(core_axis_name="c", subcore_axis_name="s")
    out_t = pl.kernel(
        _sc_body,
        out_type=jax.ShapeDtypeStruct((EMBED_DIM, BATCH), jnp.float32),
        mesh=mesh,
        scratch_types=[
            pltpu.VMEM((_BPW,), jnp.float32),
            pltpu.VMEM((_BPW,), jnp.int32),
            pltpu.VMEM((_BPW,), jnp.int32),
            pltpu.VMEM((_BPW, 128), jnp.float32),
            pltpu.VMEM((_BPW,), jnp.int32),
            pltpu.VMEM((_RING, EMBED_DIM, 128), jnp.float32),
            pltpu.VMEM((EMBED_DIM, _CHUNK), jnp.float32),
            pltpu.SemaphoreType.DMA,
        ],
        compiler_params=pltpu.CompilerParams(
            needs_layout_passes=False, use_tc_tiling_on_sc=True),
    )(inputs, bwin, tabt)
    return out_t.T


def kernel(inputs, table):
    bwin = jnp.asarray(_BWIN)
    return _embed(inputs, bwin, table)


# speedup vs baseline: 1.8577x; 1.0033x over previous
import jax
import jax.numpy as jnp
import numpy as np
from jax import lax
from jax.experimental import pallas as pl
from jax.experimental.pallas import tpu as pltpu
from jax.experimental.pallas import tpu_sc as plsc

EMBED_DIM = 32
MAX_SIZE = 1000000000
VOCAB = 1000000
BATCH = 16384

NBW = 15625
K_IDX = np.float32(np.log10(2.0) / (9.0 / (VOCAB - 1)))

_POLY = (
    np.float32(-0.024825606495141983),
    np.float32(0.2668588161468506),
    np.float32(-1.2342631816864014),
    np.float32(3.2188327312469482),
    np.float32(-5.264110565185547),
    np.float32(6.065830230712891),
    np.float32(-3.028317451477051),
)


def _window_table() -> np.ndarray:
    b = np.logspace(0.0, np.log10(MAX_SIZE), num=VOCAB).astype(np.float32)
    bpad = np.concatenate(
        [b, np.full(64 * NBW + 128 - VOCAB, np.inf, dtype=np.float32)])
    idx = 64 * np.arange(NBW)[:, None] + np.arange(128)[None, :]
    return bpad[idx]


_BWIN = _window_table()

_NC, _NS, _L = 2, 16, 16
_NW = _NC * _NS
_BPW = BATCH // _NW
_GROUPS = _BPW // _L
_CHUNK = 128
_NCHUNK = _BPW // _CHUNK
_RING = 12


def _sc_body(x_hbm, bwin_hbm, tabt_hbm, out_hbm,
             x_v, blk_v, sub_v, win_v, t_v, ring_v, obuf_v, sem, sem2):
    wid = lax.axis_index("s") * _NC + lax.axis_index("c")
    base = wid * _BPW

    pltpu.sync_copy(x_hbm.at[pl.ds(base, _BPW)], x_v)

    def guess_group(i, carry):
        x = x_v[pl.ds(i * _L, _L)]
        bits = lax.bitcast_convert_type(x, jnp.int32)
        e = (bits >> 23) - 127
        m = lax.bitcast_convert_type((bits & 0x7FFFFF) | 0x3F800000,
                                     jnp.float32)
        acc = jnp.full((_L,), _POLY[0], dtype=jnp.float32)
        for c in _POLY[1:]:
            acc = acc * m + c
        log2x = e.astype(jnp.float32) + acc
        g = (log2x * K_IDX).astype(jnp.int32) + 1
        b0 = jnp.clip(g - 8, 0, VOCAB - 17)
        blk_v[pl.ds(i * _L, _L)] = b0 >> 6
        sub_v[pl.ds(i * _L, _L)] = b0
        return carry

    lax.fori_loop(0, _GROUPS, guess_group, 0, unroll=False)

    copies = [
        pltpu.async_copy(
            bwin_hbm.at[blk_v.at[pl.ds(j * _CHUNK, _CHUNK)]],
            win_v.at[pl.ds(j * _CHUNK, _CHUNK)],
            sem,
        )
        for j in range(_NCHUNK)
    ]
    for c in copies:
        c.wait()

    def count_group(i, carry):
        x = x_v[pl.ds(i * _L, _L)]
        b0 = sub_v[pl.ds(i * _L, _L)]
        s = b0 & 63
        rows = i * _L + lax.iota(jnp.int32, _L)
        cnt = jnp.zeros((_L,), jnp.int32)
        for j in range(17):
            w = plsc.load_gather(win_v, [rows, s + j])
            cnt = cnt + jnp.where(w <= x, 1, 0)
        t = jnp.minimum(b0 + cnt, VOCAB - 1)
        t_v[pl.ds(i * _L, _L)] = t
        return carry

    lax.fori_loop(0, _GROUPS, count_group, 0, unroll=False)

    d_lo = lax.iota(jnp.int32, _L)
    d_hi = d_lo + _L

    def fetch(tv, j, slot):
        col = (tv[j] >> 7) * 128
        return pltpu.async_copy(
            tabt_hbm.at[pl.ds(0, EMBED_DIM), pl.ds(pl.multiple_of(col, 128),
                                                   128)],
            ring_v.at[slot],
            sem,
        )

    def extract(tv, j, slot, e_loc, par):
        off = jnp.full((_L,), tv[j] & 127, jnp.int32)
        ee = jnp.full((_L,), e_loc, jnp.int32)
        pp = jnp.full((_L,), par, jnp.int32)
        lo = plsc.load_gather(ring_v.at[slot], [d_lo, off])
        hi = plsc.load_gather(ring_v.at[slot], [d_hi, off])
        plsc.store_scatter(obuf_v, [pp, d_lo, ee], lo)
        plsc.store_scatter(obuf_v, [pp, d_hi, ee], hi)

    def chunk_loop(fc, carry):
        e0 = fc * _CHUNK
        par = fc & 1

        @pl.when(fc >= 2)
        def _():
            pltpu.make_async_copy(
                out_hbm.at[pl.ds(0, EMBED_DIM), pl.ds(0, _CHUNK)],
                obuf_v.at[par], sem2).wait()

        tvs = [t_v[pl.ds(e0 + g * _L, _L)] for g in range(_CHUNK // _L)]
        descs = []
        for j in range(_RING):
            descs.append(fetch(tvs[j // _L], j % _L, j % _RING))
        for j in range(_CHUNK):
            descs[j].wait()
            extract(tvs[j // _L], j % _L, j % _RING, j, par)
            if j + _RING < _CHUNK:
                jj = j + _RING
                descs.append(fetch(tvs[jj // _L], jj % _L, jj % _RING))
        pltpu.async_copy(
            obuf_v.at[par],
            out_hbm.at[pl.ds(0, EMBED_DIM),
                       pl.ds(pl.multiple_of(base + e0, _CHUNK), _CHUNK)],
            sem2)
        return carry

    lax.fori_loop(0, _NCHUNK, chunk_loop, 0, unroll=False)
    for _ in range(2):
        pltpu.make_async_copy(
            out_hbm.at[pl.ds(0, EMBED_DIM), pl.ds(0, _CHUNK)],
            obuf_v.at[0], sem2).wait()


@jax.jit
def _embed(inputs, bwin, table):
    tabt = table.T
    mesh = plsc.VectorSubcoreMesh(core_axis_name="c", subcore_axis_name="s")
    out_t = pl.kernel(
        _sc_body,
        out_type=jax.ShapeDtypeStruct((EMBED_DIM, BATCH), jnp.float32),
        mesh=mesh,
        scratch_types=[
            pltpu.VMEM((_BPW,), jnp.float32),
            pltpu.VMEM((_BPW,), jnp.int32),
            pltpu.VMEM((_BPW,), jnp.int32),
            pltpu.VMEM((_BPW, 128), jnp.float32),
            pltpu.VMEM((_BPW,), jnp.int32),
            pltpu.VMEM((_RING, EMBED_DIM, 128), jnp.float32),
            pltpu.VMEM((2, EMBED_DIM, _CHUNK), jnp.float32),
            pltpu.SemaphoreType.DMA,
            pltpu.SemaphoreType.DMA,
        ],
        compiler_params=pltpu.CompilerParams(
            needs_layout_passes=False, use_tc_tiling_on_sc=True),
    )(inputs, bwin, tabt)
    return out_t.T


def kernel(inputs, table):
    bwin = jnp.asarray(_BWIN)
    return _embed(inputs, bwin, table)
